# baseline (speedup 1.0000x reference)
import jax
import jax.numpy as jnp
from jax import lax
from jax.experimental import pallas as pl

B, SEQ, D = 4, 4096, 160
MARK_AX = 0
OP_ADD = 1
OP_SUB = 2
ALU_LO = 16
ALU_HI = 32
AX_CARRY_LO = 48
AX_CARRY_HI = 64
OUTPUT_LO = 112
OUTPUT_HI = 128

SB = 512


def _alu_body(x_ref, o_ref):
    x = x_ref[...]
    it = lax.broadcasted_iota(jnp.int32, (16, SB), 0)

    def decode(base):
        m = jnp.where(x[base:base + 16, :] > 0.5, it, 16)
        idx = jnp.min(m, axis=0, keepdims=True)
        return jnp.where(idx == 16, 0, idx)

    a_lo = decode(ALU_LO)
    a_hi = decode(ALU_HI)
    b_lo = decode(AX_CARRY_LO)
    b_hi = decode(AX_CARRY_HI)

    mark = x[MARK_AX:MARK_AX + 1, :] > 0.5
    is_add = x[OP_ADD:OP_ADD + 1, :] > 0.5
    is_sub = jnp.logical_and(jnp.logical_not(is_add), x[OP_SUB:OP_SUB + 1, :] > 0.5)
    active = jnp.logical_and(mark, jnp.logical_or(is_add, is_sub))

    sum_lo = a_lo + b_lo
    add_r_lo = jnp.bitwise_and(sum_lo, 15)
    carry = lax.shift_right_arithmetic(sum_lo, 4)
    add_r_hi = jnp.bitwise_and(a_hi + b_hi + carry, 15)

    diff_lo = a_lo - b_lo
    sub_r_lo = jnp.bitwise_and(diff_lo, 15)
    borrow = jnp.where(diff_lo < 0, 1, 0)
    sub_r_hi = jnp.bitwise_and(a_hi - b_hi - borrow, 15)

    r_lo = jnp.where(is_add, add_r_lo, sub_r_lo)
    r_hi = jnp.where(is_add, add_r_hi, sub_r_hi)

    amp = jnp.where(active, 2.0, 0.0).astype(x.dtype)
    oh_lo = jnp.where(it == r_lo, amp, 0.0)
    oh_hi = jnp.where(it == r_hi, amp, 0.0)

    o_ref[...] = x
    o_ref[OUTPUT_LO:OUTPUT_LO + 16, :] = x[OUTPUT_LO:OUTPUT_LO + 16, :] + oh_lo
    o_ref[OUTPUT_HI:OUTPUT_HI + 16, :] = x[OUTPUT_HI:OUTPUT_HI + 16, :] + oh_hi


@jax.jit
def kernel(x_bd):
    x_t = jnp.transpose(x_bd, (0, 2, 1)).reshape(B * D, SEQ)
    out_t = pl.pallas_call(
        _alu_body,
        grid=(B, SEQ // SB),
        in_specs=[pl.BlockSpec((D, SB), lambda b, s: (b, s))],
        out_specs=pl.BlockSpec((D, SB), lambda b, s: (b, s)),
        out_shape=jax.ShapeDtypeStruct((B * D, SEQ), jnp.float32),
    )(x_t)
    return jnp.transpose(out_t.reshape(B, D, SEQ), (0, 2, 1))
